# baseline (speedup 1.0000x reference)
import jax
import jax.numpy as jnp
from jax import lax
from jax.experimental import pallas as pl
from jax.experimental.pallas import tpu as pltpu
from jax.experimental.pallas import tpu_sc as plsc

_NUM_LEVELS = 8
_BASE_RES = 32
_MAX_RES = 2048
_FDIM = 8
_RES = [min(_BASE_RES * (2 ** l), _MAX_RES) for l in range(_NUM_LEVELS)]
_TAB = [min(r ** 3, 2 ** 19) for r in _RES]
_N = 262144

_NC = 2
_NS = 16
_NW = _NC * _NS
_CHUNK = _N // _NW
_BLK = 512
_NBLK = _CHUNK // _BLK
_PAD = 16

_CP = pltpu.CompilerParams(needs_layout_passes=False,
                           use_tc_tiling_on_sc=False)
_MESH = dict(core_axis_name="c", subcore_axis_name="s",
             num_cores=_NC, num_subcores=_NS)

_JPW = [t // 128 // _NW for t in _TAB]


def _wid():
    return lax.axis_index("c") * jnp.int32(_NS) + lax.axis_index("s")


def _relayout_body(Wd0, Wd1, Wd2, Wd3, Wd4, Wd5, Wd6, Wd7,
                   R0, R1, R2, R3, R4, R5, R67,
                   buf, buf2, rbuf, sem):
    Wds = [Wd0, Wd1, Wd2, Wd3, Wd4, Wd5, Wd6, Wd7]
    Rs = [R0, R1, R2, R3, R4, R5, R67]
    wid = _wid()
    lanes = lax.iota(jnp.int32, 16)
    rl = jnp.minimum(lanes, 7)
    rl2 = jnp.maximum(lanes - 8, 0)
    mask_lo = lanes < 8
    mask_hi = lanes >= 8

    def do_table(t, pair):
        jpw = _JPW[t]
        cj = min(16, jpw)
        nch = jpw // cj
        Wd = Wds[t]
        R = Rs[t] if not pair else Rs[6]

        def chunk_body(ck, carry):
            j0 = wid * jnp.int32(jpw) + ck * jnp.int32(cj)
            pltpu.sync_copy(Wd.at[pl.ds(j0, cj)], buf.at[pl.ds(0, cj)])
            if pair:
                pltpu.sync_copy(Wds[7].at[pl.ds(j0, cj)],
                                buf2.at[pl.ds(0, cj)])

            def jj_body(jj, carry2):
                jjv = jnp.full((16,), 0, jnp.int32) + jj

                def cc_body(cc, carry3):
                    c8 = cc * jnp.int32(8)
                    cb = jnp.full((16,), 0, jnp.int32) + c8
                    o8 = jj * jnp.int32(2048) + cc * jnp.int32(128)
                    for u in range(8):
                        cvec = cb + jnp.int32(u)
                        g = plsc.load_gather(buf, [jjv, rl, cvec],
                                             mask=mask_lo)
                        if pair:
                            g2 = plsc.load_gather(buf2, [jjv, rl2, cvec],
                                                  mask=mask_hi)
                            g = jnp.where(mask_lo, g, g2)
                        rbuf[pl.ds(o8 + jnp.int32(u * 16), 16)] = g
                    return carry3

                lax.fori_loop(jnp.int32(0), jnp.int32(16), cc_body, carry2)
                return carry2

            lax.fori_loop(jnp.int32(0), jnp.int32(cj), jj_body, jnp.int32(0))
            rowoff = (wid * jnp.int32(jpw) + ck * jnp.int32(cj)) * jnp.int32(2048)
            pltpu.sync_copy(rbuf.at[pl.ds(0, cj * 2048)],
                            R.at[pl.ds(rowoff, cj * 2048)])
            return carry

        lax.fori_loop(jnp.int32(0), jnp.int32(nch), chunk_body, jnp.int32(0))

    for t in range(6):
        do_table(t, pair=False)
    do_table(6, pair=True)


def _lookup_body(posT, R0, R1, R2, R3, R4, R5, R67, out2,
                 posv, idxv, rows, ostage, sem, sem2):
    Rs = [R0, R1, R2, R3, R4, R5, R67]
    wid = _wid()
    base = wid * jnp.int32(_CHUNK)
    lanes = lax.iota(jnp.int32, 16)

    def blk_body(b, carry):
        row0 = base + b * jnp.int32(_BLK)
        for f in range(3):
            pltpu.sync_copy(posT.at[jnp.int32(f), pl.ds(row0, _BLK)],
                            posv.at[jnp.int32(f)])

        def cmp_body(i, carry2):
            i16 = i * jnp.int32(16)
            x = posv[jnp.int32(0), pl.ds(i16, 16)]
            y = posv[jnp.int32(1), pl.ds(i16, 16)]
            z = posv[jnp.int32(2), pl.ds(i16, 16)]
            ux = (x + 1.0) * 0.5
            uy = (y + 1.0) * 0.5
            uz = (z + 1.0) * 0.5
            for l in range(7):
                r = float(_RES[l])
                hi = r - 1.0
                px = jnp.minimum(jnp.maximum(ux * r, 0.0), hi)
                py = jnp.minimum(jnp.maximum(uy * r, 0.0), hi)
                pz = jnp.minimum(jnp.maximum(uz * r, 0.0), hi)
                idxf = px * (r * r) + py * r + pz
                idxf = jnp.minimum(idxf, float(_TAB[l] - 1))
                idxv[jnp.int32(l), pl.ds(i16, 16)] = idxf.astype(jnp.int32)
            return carry2

        lax.fori_loop(jnp.int32(0), jnp.int32(_BLK // 16), cmp_body,
                      jnp.int32(0))

        descs = []
        for l in range(7):
            d = pltpu.make_async_copy(Rs[l].at[idxv.at[jnp.int32(l)]],
                                      rows.at[jnp.int32(l)], sem)
            d.start()
            descs.append(d)
        for d in descs:
            d.wait()

        wdescs = []
        for l in range(_NUM_LEVELS):
            l7 = jnp.full((16,), 0, jnp.int32) + jnp.int32(min(l, 6))
            coff = 0 if l < 7 else 8

            def tj_body(tj, carry3, l=l, l7=l7, coff=coff):
                for cb in range(8):
                    pvec = lanes + (tj * jnp.int32(128) + jnp.int32(cb * 16))
                    for r in range(8):
                        cvec = jnp.full((16,), 0, jnp.int32) + jnp.int32(coff + r)
                        g = plsc.load_gather(rows, [l7, pvec, cvec])
                        ostage[jnp.int32(l), tj, jnp.int32(r),
                               pl.ds(cb * 16, 16)] = g
                return carry3

            lax.fori_loop(jnp.int32(0), jnp.int32(_BLK // 128), tj_body,
                          jnp.int32(0))
            tj0 = wid * jnp.int32(_CHUNK // 128) + b * jnp.int32(_BLK // 128)
            d = pltpu.make_async_copy(
                ostage.at[jnp.int32(l)],
                out2.at[jnp.int32(l), pl.ds(tj0, _BLK // 128)], sem2)
            d.start()
            wdescs.append(d)
        for d in wdescs:
            d.wait()
        return carry

    lax.fori_loop(jnp.int32(0), jnp.int32(_NBLK), blk_body, jnp.int32(0))


@jax.jit
def kernel(positions, W0, W1, W2, W3, W4, W5, W6, W7):
    Ws = [W0, W1, W2, W3, W4, W5, W6, W7]
    Wds = [W.reshape(_TAB[i] // 128, 128, 8).transpose(0, 2, 1)
           for i, W in enumerate(Ws)]
    posT = positions.T

    mesh = plsc.VectorSubcoreMesh(**_MESH)

    relayout = pl.kernel(
        _relayout_body,
        out_type=tuple(
            jax.ShapeDtypeStruct((_TAB[i] * _PAD,), jnp.float32)
            for i in range(7)),
        mesh=mesh,
        scratch_types=[
            pltpu.VMEM((16, 8, 128), jnp.float32),
            pltpu.VMEM((16, 8, 128), jnp.float32),
            pltpu.VMEM((2048 * _PAD,), jnp.float32),
            pltpu.SemaphoreType.DMA,
        ],
        compiler_params=_CP,
    )
    Rs = [r.reshape(-1, _PAD) for r in relayout(*Wds)]

    lookup = pl.kernel(
        _lookup_body,
        out_type=jax.ShapeDtypeStruct((8, _N // 128, 8, 128), jnp.float32),
        mesh=mesh,
        scratch_types=[
            pltpu.VMEM((3, _BLK), jnp.float32),
            pltpu.VMEM((7, _BLK), jnp.int32),
            pltpu.VMEM((7, _BLK, _PAD), jnp.float32),
            pltpu.VMEM((_NUM_LEVELS, _BLK // 128, 8, 128), jnp.float32),
            pltpu.SemaphoreType.DMA,
            pltpu.SemaphoreType.DMA,
        ],
        compiler_params=_CP,
    )
    out2 = lookup(posT, *Rs)

    return out2.transpose(1, 3, 0, 2).reshape(_N, _NUM_LEVELS * _FDIM)


# speedup vs baseline: 1.1639x; 1.1639x over previous
import jax
import jax.numpy as jnp
from jax import lax
from jax.experimental import pallas as pl
from jax.experimental.pallas import tpu as pltpu
from jax.experimental.pallas import tpu_sc as plsc

_NUM_LEVELS = 8
_BASE_RES = 32
_MAX_RES = 2048
_FDIM = 8
_RES = [min(_BASE_RES * (2 ** l), _MAX_RES) for l in range(_NUM_LEVELS)]
_TAB = [min(r ** 3, 2 ** 19) for r in _RES]
_N = 262144

_NC = 2
_NS = 16
_NW = _NC * _NS
_CHUNK = _N // _NW
_BLK = 512
_NBLK = _CHUNK // _BLK

_CP = pltpu.CompilerParams(needs_layout_passes=False,
                           use_tc_tiling_on_sc=False)
_MESH = dict(core_axis_name="c", subcore_axis_name="s",
             num_cores=_NC, num_subcores=_NS)

_JPW = [t // 128 // _NW for t in _TAB]


def _wid():
    return lax.axis_index("c") * jnp.int32(_NS) + lax.axis_index("s")


def _relayout_body(Wd0, Wd1, Wd2, Wd3, Wd4, Wd5, Wd6, Wd7,
                   R0, R1, R2, R3, R4, R5, R67,
                   buf, buf2, rbuf, sem):
    Wds = [Wd0, Wd1, Wd2, Wd3, Wd4, Wd5, Wd6, Wd7]
    Rs = [R0, R1, R2, R3, R4, R5, R67]
    wid = _wid()
    lanes = lax.iota(jnp.int32, 16)
    rmod = lanes & jnp.int32(7)
    choff = lanes >> jnp.int32(3)
    rl2 = jnp.maximum(lanes - 8, 0)
    mask_lo = lanes < 8
    mask_hi = lanes >= 8

    def do_table(t, pair):
        jpw = _JPW[t]
        cj = min(16, jpw)
        nch = jpw // cj
        Wd = Wds[t]
        R = Rs[t] if not pair else Rs[6]
        fpj = 2048 if pair else 1024

        def chunk_body(ck, carry):
            j0 = wid * jnp.int32(jpw) + ck * jnp.int32(cj)
            pltpu.sync_copy(Wd.at[pl.ds(j0, cj)], buf.at[pl.ds(0, cj)])
            if pair:
                pltpu.sync_copy(Wds[7].at[pl.ds(j0, cj)],
                                buf2.at[pl.ds(0, cj)])

            def jj_body(jj, carry2):
                jjv = jnp.full((16,), 0, jnp.int32) + jj
                o0 = jj * jnp.int32(fpj)
                if not pair:
                    for k in range(64):
                        cvec = choff + jnp.int32(2 * k)
                        g = plsc.load_gather(buf, [jjv, rmod, cvec])
                        rbuf[pl.ds(o0 + jnp.int32(16 * k), 16)] = g
                else:
                    for c in range(128):
                        cvec = jnp.full((16,), c, jnp.int32)
                        g = plsc.load_gather(buf, [jjv, rmod, cvec],
                                             mask=mask_lo)
                        g2 = plsc.load_gather(buf2, [jjv, rl2, cvec],
                                              mask=mask_hi)
                        g = jnp.where(mask_lo, g, g2)
                        rbuf[pl.ds(o0 + jnp.int32(16 * c), 16)] = g
                return carry2

            lax.fori_loop(jnp.int32(0), jnp.int32(cj), jj_body, jnp.int32(0))
            off = (wid * jnp.int32(jpw) + ck * jnp.int32(cj)) * jnp.int32(fpj)
            pltpu.sync_copy(rbuf.at[pl.ds(0, cj * fpj)],
                            R.at[pl.ds(off, cj * fpj)])
            return carry

        lax.fori_loop(jnp.int32(0), jnp.int32(nch), chunk_body, jnp.int32(0))

    for t in range(6):
        do_table(t, pair=False)
    do_table(6, pair=True)


def _lookup_body(posT, R0, R1, R2, R3, R4, R5, R67, out2,
                 posv, idxv, rows, rows67, ostage, sem, sem2):
    Rs = [R0, R1, R2, R3, R4, R5]
    wid = _wid()
    base = wid * jnp.int32(_CHUNK)
    lanes = lax.iota(jnp.int32, 16)

    def blk_body(b, carry):
        row0 = base + b * jnp.int32(_BLK)
        for f in range(3):
            pltpu.sync_copy(posT.at[jnp.int32(f), pl.ds(row0, _BLK)],
                            posv.at[jnp.int32(f)])

        def cmp_body(i, carry2):
            i16 = i * jnp.int32(16)
            x = posv[jnp.int32(0), pl.ds(i16, 16)]
            y = posv[jnp.int32(1), pl.ds(i16, 16)]
            z = posv[jnp.int32(2), pl.ds(i16, 16)]
            ux = (x + 1.0) * 0.5
            uy = (y + 1.0) * 0.5
            uz = (z + 1.0) * 0.5
            for l in range(7):
                r = float(_RES[l])
                hi = r - 1.0
                px = jnp.minimum(jnp.maximum(ux * r, 0.0), hi)
                py = jnp.minimum(jnp.maximum(uy * r, 0.0), hi)
                pz = jnp.minimum(jnp.maximum(uz * r, 0.0), hi)
                idxf = px * (r * r) + py * r + pz
                idxf = jnp.minimum(idxf, float(_TAB[l] - 1))
                idxv[jnp.int32(l), pl.ds(i16, 16)] = idxf.astype(jnp.int32)
            return carry2

        lax.fori_loop(jnp.int32(0), jnp.int32(_BLK // 16), cmp_body,
                      jnp.int32(0))

        descs = []
        for l in range(6):
            d = pltpu.make_async_copy(Rs[l].at[idxv.at[jnp.int32(l)]],
                                      rows.at[jnp.int32(l)], sem)
            d.start()
            descs.append(d)
        d = pltpu.make_async_copy(R67.at[idxv.at[jnp.int32(6)]], rows67, sem)
        d.start()
        descs.append(d)
        for d in descs:
            d.wait()

        wdescs = []
        for l in range(_NUM_LEVELS):
            if l < 6:
                l7 = jnp.full((16,), 0, jnp.int32) + jnp.int32(l)

            def tj_body(tj, carry3, l=l):
                for cb in range(8):
                    pvec = lanes + (tj * jnp.int32(128) + jnp.int32(cb * 16))
                    for r in range(8):
                        if l < 6:
                            cvec = jnp.full((16,), r, jnp.int32)
                            g = plsc.load_gather(
                                rows,
                                [jnp.full((16,), l, jnp.int32), pvec, cvec])
                        else:
                            cvec = jnp.full((16,), (l - 6) * 8 + r, jnp.int32)
                            g = plsc.load_gather(rows67, [pvec, cvec])
                        ostage[jnp.int32(l), tj, jnp.int32(r),
                               pl.ds(cb * 16, 16)] = g
                return carry3

            lax.fori_loop(jnp.int32(0), jnp.int32(_BLK // 128), tj_body,
                          jnp.int32(0))
            tj0 = wid * jnp.int32(_CHUNK // 128) + b * jnp.int32(_BLK // 128)
            d = pltpu.make_async_copy(
                ostage.at[jnp.int32(l)],
                out2.at[jnp.int32(l), pl.ds(tj0, _BLK // 128)], sem2)
            d.start()
            wdescs.append(d)
        for d in wdescs:
            d.wait()
        return carry

    lax.fori_loop(jnp.int32(0), jnp.int32(_NBLK), blk_body, jnp.int32(0))


@jax.jit
def kernel(positions, W0, W1, W2, W3, W4, W5, W6, W7):
    Ws = [W0, W1, W2, W3, W4, W5, W6, W7]
    Wds = [W.reshape(_TAB[i] // 128, 128, 8).transpose(0, 2, 1)
           for i, W in enumerate(Ws)]
    posT = positions.T

    mesh = plsc.VectorSubcoreMesh(**_MESH)

    relayout = pl.kernel(
        _relayout_body,
        out_type=tuple(
            jax.ShapeDtypeStruct((_TAB[i] * _FDIM,), jnp.float32)
            for i in range(6)) + (
            jax.ShapeDtypeStruct((_TAB[6] * 16,), jnp.float32),),
        mesh=mesh,
        scratch_types=[
            pltpu.VMEM((16, 8, 128), jnp.float32),
            pltpu.VMEM((16, 8, 128), jnp.float32),
            pltpu.VMEM((16 * 2048,), jnp.float32),
            pltpu.SemaphoreType.DMA,
        ],
        compiler_params=_CP,
    )
    Rflat = relayout(*Wds)
    Rs = [Rflat[i].reshape(_TAB[i], _FDIM) for i in range(6)]
    R67 = Rflat[6].reshape(_TAB[6], 16)

    lookup = pl.kernel(
        _lookup_body,
        out_type=jax.ShapeDtypeStruct((8, _N // 128, 8, 128), jnp.float32),
        mesh=mesh,
        scratch_types=[
            pltpu.VMEM((3, _BLK), jnp.float32),
            pltpu.VMEM((7, _BLK), jnp.int32),
            pltpu.VMEM((6, _BLK, _FDIM), jnp.float32),
            pltpu.VMEM((_BLK, 16), jnp.float32),
            pltpu.VMEM((_NUM_LEVELS, _BLK // 128, 8, 128), jnp.float32),
            pltpu.SemaphoreType.DMA,
            pltpu.SemaphoreType.DMA,
        ],
        compiler_params=_CP,
    )
    out2 = lookup(posT, *Rs, R67)

    return out2.transpose(1, 3, 0, 2).reshape(_N, _NUM_LEVELS * _FDIM)


# speedup vs baseline: 2.1292x; 1.8293x over previous
import jax
import jax.numpy as jnp
from jax import lax
from jax.experimental import pallas as pl
from jax.experimental.pallas import tpu as pltpu
from jax.experimental.pallas import tpu_sc as plsc

_NUM_LEVELS = 8
_BASE_RES = 32
_MAX_RES = 2048
_FDIM = 8
_RES = [min(_BASE_RES * (2 ** l), _MAX_RES) for l in range(_NUM_LEVELS)]
_TAB = [min(r ** 3, 2 ** 19) for r in _RES]
_N = 262144

_NC = 2
_NS = 16
_NW = _NC * _NS
_CHUNK = _N // _NW
_BLK = 512
_NBLK = _CHUNK // _BLK

_CP = pltpu.CompilerParams(needs_layout_passes=False,
                           use_tc_tiling_on_sc=False)
_MESH = dict(core_axis_name="c", subcore_axis_name="s",
             num_cores=_NC, num_subcores=_NS)

_JPW = [t // 128 // _NW for t in _TAB]


def _wid():
    return lax.axis_index("c") * jnp.int32(_NS) + lax.axis_index("s")


def _relayout_body(Wd0, Wd1, Wd2, Wd3, Wd4, Wd5, Wd6, Wd7,
                   R0, R1, R2, R3, R4, R5, R67,
                   buf, buf2, rbuf, sem):
    Wds = [Wd0, Wd1, Wd2, Wd3, Wd4, Wd5, Wd6, Wd7]
    Rs = [R0, R1, R2, R3, R4, R5, R67]
    wid = _wid()
    lanes = lax.iota(jnp.int32, 16)
    rmod = lanes & jnp.int32(7)
    choff = lanes >> jnp.int32(3)
    rl2 = jnp.maximum(lanes - 8, 0)
    mask_lo = lanes < 8
    mask_hi = lanes >= 8

    def do_table(t, pair):
        jpw = _JPW[t]
        cj = min(16, jpw)
        nch = jpw // cj
        Wd = Wds[t]
        R = Rs[t] if not pair else Rs[6]
        fpj = 2048 if pair else 1024

        def chunk_body(ck, carry):
            j0 = wid * jnp.int32(jpw) + ck * jnp.int32(cj)
            pltpu.sync_copy(Wd.at[pl.ds(j0, cj)], buf.at[pl.ds(0, cj)])
            if pair:
                pltpu.sync_copy(Wds[7].at[pl.ds(j0, cj)],
                                buf2.at[pl.ds(0, cj)])

            def jj_body(jj, carry2):
                jjv = jnp.full((16,), 0, jnp.int32) + jj
                o0 = jj * jnp.int32(fpj)
                if not pair:
                    for k in range(64):
                        cvec = choff + jnp.int32(2 * k)
                        g = plsc.load_gather(buf, [jjv, rmod, cvec])
                        rbuf[pl.ds(o0 + jnp.int32(16 * k), 16)] = g
                else:
                    for c in range(128):
                        cvec = jnp.full((16,), c, jnp.int32)
                        g = plsc.load_gather(buf, [jjv, rmod, cvec],
                                             mask=mask_lo)
                        g2 = plsc.load_gather(buf2, [jjv, rl2, cvec],
                                              mask=mask_hi)
                        g = jnp.where(mask_lo, g, g2)
                        rbuf[pl.ds(o0 + jnp.int32(16 * c), 16)] = g
                return carry2

            lax.fori_loop(jnp.int32(0), jnp.int32(cj), jj_body, jnp.int32(0))
            off = (wid * jnp.int32(jpw) + ck * jnp.int32(cj)) * jnp.int32(fpj)
            pltpu.sync_copy(rbuf.at[pl.ds(0, cj * fpj)],
                            R.at[pl.ds(off, cj * fpj)])
            return carry

        lax.fori_loop(jnp.int32(0), jnp.int32(nch), chunk_body, jnp.int32(0))

    for t in range(6):
        do_table(t, pair=False)
    do_table(6, pair=True)


_CSLOT = 512


def _lookup_body(posT, R0, R1, R2, R3, R4, R5, R67, out2,
                 posv, idxv, cidxv, ptrv, rows01, rowsC, rows67C,
                 idxcache, ostage, sem, sem2):
    Rs = [R0, R1, R2, R3, R4, R5]
    wid = _wid()
    base = wid * jnp.int32(_CHUNK)
    lanes = lax.iota(jnp.int32, 16)
    z16 = jnp.zeros((16,), jnp.int32)

    idxcache[pl.ds(0, 16)] = z16 + jnp.int32(_TAB[2] - 1)
    cdescs = []
    for lc in range(4):
        d = pltpu.make_async_copy(
            Rs[lc + 2].at[idxcache.at[pl.ds(0, 8)]],
            rowsC.at[jnp.int32(lc), pl.ds(_CSLOT, 8)], sem)
        d.start()
        cdescs.append(d)
    d = pltpu.make_async_copy(R67.at[idxcache.at[pl.ds(0, 8)]],
                              rows67C.at[pl.ds(_CSLOT, 8)], sem)
    d.start()
    cdescs.append(d)
    for d in cdescs:
        d.wait()

    def chunk_copy(lc, ck):
        cs = ck * jnp.int32(128)
        if lc < 4:
            return pltpu.make_async_copy(
                Rs[lc + 2].at[cidxv.at[jnp.int32(lc), pl.ds(cs, 128)]],
                rowsC.at[jnp.int32(lc), pl.ds(cs, 128)], sem)
        return pltpu.make_async_copy(
            R67.at[cidxv.at[jnp.int32(4), pl.ds(cs, 128)]],
            rows67C.at[pl.ds(cs, 128)], sem)

    def blk_body(b, carry):
        row0 = base + b * jnp.int32(_BLK)
        for f in range(3):
            pltpu.sync_copy(posT.at[jnp.int32(f), pl.ds(row0, _BLK)],
                            posv.at[jnp.int32(f)])

        def zero_body(i, carry2):
            i16 = i * jnp.int32(16)
            for lc in range(5):
                cidxv[jnp.int32(lc), pl.ds(i16, 16)] = z16
            return carry2

        lax.fori_loop(jnp.int32(0), jnp.int32(_BLK // 16), zero_body,
                      jnp.int32(0))

        def cmp_body(i, offs):
            offs = list(offs)
            i16 = i * jnp.int32(16)
            x = posv[jnp.int32(0), pl.ds(i16, 16)]
            y = posv[jnp.int32(1), pl.ds(i16, 16)]
            z = posv[jnp.int32(2), pl.ds(i16, 16)]
            ux = (x + 1.0) * 0.5
            uy = (y + 1.0) * 0.5
            uz = (z + 1.0) * 0.5
            for l in range(7):
                r = float(_RES[l])
                hi = r - 1.0
                px = jnp.minimum(jnp.maximum(ux * r, 0.0), hi)
                py = jnp.minimum(jnp.maximum(uy * r, 0.0), hi)
                pz = jnp.minimum(jnp.maximum(uz * r, 0.0), hi)
                idxf = px * (r * r) + py * r + pz
                idxf = jnp.minimum(idxf, float(_TAB[l] - 1))
                idx = idxf.astype(jnp.int32)
                if l < 2:
                    idxv[jnp.int32(l), pl.ds(i16, 16)] = idx
                else:
                    lc = l - 2
                    m = idx < jnp.int32(_TAB[l] - 1)
                    mi = m.astype(jnp.int32)
                    excl = plsc.cumsum(mi) - mi
                    dst = offs[lc] + excl
                    plsc.store_scatter(
                        cidxv, [jnp.full((16,), lc, jnp.int32), dst],
                        idx, mask=m)
                    ptrv[jnp.int32(lc), pl.ds(i16, 16)] = (
                        jnp.where(m, dst, jnp.int32(_CSLOT)))
                    offs[lc] = offs[lc] + plsc.all_reduce_population_count(m)
            return tuple(offs)

        offs = lax.fori_loop(
            jnp.int32(0), jnp.int32(_BLK // 16), cmp_body,
            tuple(z16 for _ in range(5)))
        plsc.subcore_barrier()
        cnts = [jnp.max(offs[k]) for k in range(5)]

        for l in range(2):
            pltpu.make_async_copy(Rs[l].at[idxv.at[jnp.int32(l)]],
                                  rows01.at[jnp.int32(l)], sem).start()
        nchs = []
        for lc in range(5):
            nch = lax.shift_right_logical(cnts[lc] + jnp.int32(127),
                                          jnp.int32(7))
            nchs.append(nch)

            def fire(ck, carry2, lc=lc):
                chunk_copy(lc, ck).start()
                return carry2

            lax.fori_loop(jnp.int32(0), nch, fire, jnp.int32(0))

        for l in range(2):
            pltpu.make_async_copy(Rs[l].at[idxv.at[jnp.int32(l)]],
                                  rows01.at[jnp.int32(l)], sem).wait()
        for lc in range(5):
            def drain(ck, carry2, lc=lc):
                chunk_copy(lc, ck).wait()
                return carry2

            lax.fori_loop(jnp.int32(0), nchs[lc], drain, jnp.int32(0))

        wdescs = []
        for l in range(_NUM_LEVELS):

            def tj_body(tj, carry3, l=l):
                for cb in range(8):
                    p0 = tj * jnp.int32(128) + jnp.int32(cb * 16)
                    if l < 2:
                        pv = lanes + p0
                    elif l < 6:
                        pv = ptrv[jnp.int32(l - 2), pl.ds(p0, 16)]
                    else:
                        pv = ptrv[jnp.int32(4), pl.ds(p0, 16)]
                    for r in range(8):
                        if l < 2:
                            g = plsc.load_gather(
                                rows01,
                                [jnp.full((16,), l, jnp.int32), pv,
                                 jnp.full((16,), r, jnp.int32)])
                        elif l < 6:
                            g = plsc.load_gather(
                                rowsC,
                                [jnp.full((16,), l - 2, jnp.int32), pv,
                                 jnp.full((16,), r, jnp.int32)])
                        else:
                            g = plsc.load_gather(
                                rows67C,
                                [pv, jnp.full((16,), (l - 6) * 8 + r,
                                              jnp.int32)])
                        ostage[jnp.int32(l), tj, jnp.int32(r),
                               pl.ds(cb * 16, 16)] = g
                return carry3

            lax.fori_loop(jnp.int32(0), jnp.int32(_BLK // 128), tj_body,
                          jnp.int32(0))
            tj0 = wid * jnp.int32(_CHUNK // 128) + b * jnp.int32(_BLK // 128)
            d = pltpu.make_async_copy(
                ostage.at[jnp.int32(l)],
                out2.at[jnp.int32(l), pl.ds(tj0, _BLK // 128)], sem2)
            d.start()
            wdescs.append(d)
        for d in wdescs:
            d.wait()
        return carry

    lax.fori_loop(jnp.int32(0), jnp.int32(_NBLK), blk_body, jnp.int32(0))


@jax.jit
def kernel(positions, W0, W1, W2, W3, W4, W5, W6, W7):
    Ws = [W0, W1, W2, W3, W4, W5, W6, W7]
    Wds = [W.reshape(_TAB[i] // 128, 128, 8).transpose(0, 2, 1)
           for i, W in enumerate(Ws)]
    posT = positions.T

    mesh = plsc.VectorSubcoreMesh(**_MESH)

    relayout = pl.kernel(
        _relayout_body,
        out_type=tuple(
            jax.ShapeDtypeStruct((_TAB[i] * _FDIM,), jnp.float32)
            for i in range(6)) + (
            jax.ShapeDtypeStruct((_TAB[6] * 16,), jnp.float32),),
        mesh=mesh,
        scratch_types=[
            pltpu.VMEM((16, 8, 128), jnp.float32),
            pltpu.VMEM((16, 8, 128), jnp.float32),
            pltpu.VMEM((16 * 2048,), jnp.float32),
            pltpu.SemaphoreType.DMA,
        ],
        compiler_params=_CP,
    )
    Rflat = relayout(*Wds)
    Rs = [Rflat[i].reshape(_TAB[i], _FDIM) for i in range(6)]
    R67 = Rflat[6].reshape(_TAB[6], 16)

    lookup = pl.kernel(
        _lookup_body,
        out_type=jax.ShapeDtypeStruct((8, _N // 128, 8, 128), jnp.float32),
        mesh=mesh,
        scratch_types=[
            pltpu.VMEM((3, _BLK), jnp.float32),
            pltpu.VMEM((2, _BLK), jnp.int32),
            pltpu.VMEM((5, _BLK), jnp.int32),
            pltpu.VMEM((5, _BLK), jnp.int32),
            pltpu.VMEM((2, _BLK, _FDIM), jnp.float32),
            pltpu.VMEM((4, _CSLOT + 16, _FDIM), jnp.float32),
            pltpu.VMEM((_CSLOT + 16, 16), jnp.float32),
            pltpu.VMEM((16,), jnp.int32),
            pltpu.VMEM((_NUM_LEVELS, _BLK // 128, 8, 128), jnp.float32),
            pltpu.SemaphoreType.DMA,
            pltpu.SemaphoreType.DMA,
        ],
        compiler_params=_CP,
    )
    out2 = lookup(posT, *Rs, R67)

    return out2.transpose(1, 3, 0, 2).reshape(_N, _NUM_LEVELS * _FDIM)


# speedup vs baseline: 2.2688x; 1.0656x over previous
import jax
import jax.numpy as jnp
from jax import lax
from jax.experimental import pallas as pl
from jax.experimental.pallas import tpu as pltpu
from jax.experimental.pallas import tpu_sc as plsc

_NUM_LEVELS = 8
_BASE_RES = 32
_MAX_RES = 2048
_FDIM = 8
_RES = [min(_BASE_RES * (2 ** l), _MAX_RES) for l in range(_NUM_LEVELS)]
_TAB = [min(r ** 3, 2 ** 19) for r in _RES]
_N = 262144

_NC = 2
_NS = 16
_NW = _NC * _NS
_CHUNK = _N // _NW
_BLK = 512
_NBLK = _CHUNK // _BLK

_CP = pltpu.CompilerParams(needs_layout_passes=False,
                           use_tc_tiling_on_sc=False)
_MESH = dict(core_axis_name="c", subcore_axis_name="s",
             num_cores=_NC, num_subcores=_NS)

_JPW = [t // 128 // _NW for t in _TAB]


def _wid():
    return lax.axis_index("c") * jnp.int32(_NS) + lax.axis_index("s")


def _relayout_body(Wd0, Wd1, Wd2, Wd3, Wd4, Wd5, Wd6, Wd7,
                   R0, R1, R2, R3, R4, R5, R67,
                   buf, buf2, rbuf, sem):
    Wds = [Wd0, Wd1, Wd2, Wd3, Wd4, Wd5, Wd6, Wd7]
    Rs = [R0, R1, R2, R3, R4, R5, R67]
    wid = _wid()
    lanes = lax.iota(jnp.int32, 16)
    rmod = lanes & jnp.int32(7)
    choff = lanes >> jnp.int32(3)
    rl2 = jnp.maximum(lanes - 8, 0)
    mask_lo = lanes < 8
    mask_hi = lanes >= 8

    def do_table(t, pair):
        jpw = _JPW[t]
        cj = min(16, jpw)
        nch = jpw // cj
        Wd = Wds[t]
        R = Rs[t] if not pair else Rs[6]
        fpj = 2048 if pair else 1024

        def read_descs(ck, pp):
            j0 = wid * jnp.int32(jpw) + ck * jnp.int32(cj)
            ds = [pltpu.make_async_copy(
                Wd.at[pl.ds(j0, cj)], buf.at[pp, pl.ds(0, cj)], sem)]
            if pair:
                ds.append(pltpu.make_async_copy(
                    Wds[7].at[pl.ds(j0, cj)], buf2.at[pp, pl.ds(0, cj)], sem))
            return ds

        for d in read_descs(jnp.int32(0), jnp.int32(0)):
            d.start()

        def chunk_body(ck, carry):
            pp = ck & jnp.int32(1)
            for d in read_descs(ck, pp):
                d.wait()
            nxt = jnp.minimum(ck + jnp.int32(1), jnp.int32(nch - 1))
            for d in read_descs(nxt, pp ^ jnp.int32(1)):
                d.start()
            ppv = jnp.full((16,), 0, jnp.int32) + pp

            def jj_body(jj, carry2):
                jjv = jnp.full((16,), 0, jnp.int32) + jj
                o0 = jj * jnp.int32(fpj)
                if not pair:
                    for k in range(64):
                        cvec = choff + jnp.int32(2 * k)
                        g = plsc.load_gather(buf, [ppv, jjv, rmod, cvec])
                        rbuf[pl.ds(o0 + jnp.int32(16 * k), 16)] = g
                else:
                    for c in range(128):
                        cvec = jnp.full((16,), c, jnp.int32)
                        g = plsc.load_gather(buf, [ppv, jjv, rmod, cvec],
                                             mask=mask_lo)
                        g2 = plsc.load_gather(buf2, [ppv, jjv, rl2, cvec],
                                              mask=mask_hi)
                        g = jnp.where(mask_lo, g, g2)
                        rbuf[pl.ds(o0 + jnp.int32(16 * c), 16)] = g
                return carry2

            lax.fori_loop(jnp.int32(0), jnp.int32(cj), jj_body, jnp.int32(0))
            off = (wid * jnp.int32(jpw) + ck * jnp.int32(cj)) * jnp.int32(fpj)
            pltpu.sync_copy(rbuf.at[pl.ds(0, cj * fpj)],
                            R.at[pl.ds(off, cj * fpj)])
            return carry

        lax.fori_loop(jnp.int32(0), jnp.int32(nch), chunk_body, jnp.int32(0))
        for d in read_descs(jnp.int32(nch - 1), jnp.int32(nch & 1)):
            d.wait()

    for t in range(6):
        do_table(t, pair=False)
    do_table(6, pair=True)


_CSLOT = 512


def _lookup_body(posT, R0, R1, R2, R3, R4, R5, R67, out2,
                 posv, idxv, cidxv, ptrv, rows01, rowsC, rows67C,
                 idxcache, ostage, sem, sem2):
    Rs = [R0, R1, R2, R3, R4, R5]
    wid = _wid()
    base = wid * jnp.int32(_CHUNK)
    lanes = lax.iota(jnp.int32, 16)
    z16 = jnp.zeros((16,), jnp.int32)

    idxcache[pl.ds(0, 16)] = z16 + jnp.int32(_TAB[2] - 1)
    cdescs = []
    for lc in range(4):
        d = pltpu.make_async_copy(
            Rs[lc + 2].at[idxcache.at[pl.ds(0, 8)]],
            rowsC.at[jnp.int32(lc), pl.ds(_CSLOT, 8)], sem)
        d.start()
        cdescs.append(d)
    d = pltpu.make_async_copy(R67.at[idxcache.at[pl.ds(0, 8)]],
                              rows67C.at[pl.ds(_CSLOT, 8)], sem)
    d.start()
    cdescs.append(d)
    for d in cdescs:
        d.wait()

    def chunk_copy(lc, ck):
        cs = ck * jnp.int32(128)
        if lc < 4:
            return pltpu.make_async_copy(
                Rs[lc + 2].at[cidxv.at[jnp.int32(lc), pl.ds(cs, 128)]],
                rowsC.at[jnp.int32(lc), pl.ds(cs, 128)], sem)
        return pltpu.make_async_copy(
            R67.at[cidxv.at[jnp.int32(4), pl.ds(cs, 128)]],
            rows67C.at[pl.ds(cs, 128)], sem)

    def blk_body(b, carry):
        row0 = base + b * jnp.int32(_BLK)
        for f in range(3):
            pltpu.sync_copy(posT.at[jnp.int32(f), pl.ds(row0, _BLK)],
                            posv.at[jnp.int32(f)])

        def zero_body(i, carry2):
            i16 = i * jnp.int32(16)
            for lc in range(5):
                cidxv[jnp.int32(lc), pl.ds(i16, 16)] = z16
            return carry2

        lax.fori_loop(jnp.int32(0), jnp.int32(_BLK // 16), zero_body,
                      jnp.int32(0))

        def cmp_body(i, offs):
            offs = list(offs)
            i16 = i * jnp.int32(16)
            x = posv[jnp.int32(0), pl.ds(i16, 16)]
            y = posv[jnp.int32(1), pl.ds(i16, 16)]
            z = posv[jnp.int32(2), pl.ds(i16, 16)]
            ux = (x + 1.0) * 0.5
            uy = (y + 1.0) * 0.5
            uz = (z + 1.0) * 0.5
            for l in range(7):
                r = float(_RES[l])
                hi = r - 1.0
                px = jnp.minimum(jnp.maximum(ux * r, 0.0), hi)
                py = jnp.minimum(jnp.maximum(uy * r, 0.0), hi)
                pz = jnp.minimum(jnp.maximum(uz * r, 0.0), hi)
                idxf = px * (r * r) + py * r + pz
                idxf = jnp.minimum(idxf, float(_TAB[l] - 1))
                idx = idxf.astype(jnp.int32)
                if l < 2:
                    idxv[jnp.int32(l), pl.ds(i16, 16)] = idx
                else:
                    lc = l - 2
                    m = idx < jnp.int32(_TAB[l] - 1)
                    mi = m.astype(jnp.int32)
                    excl = plsc.cumsum(mi) - mi
                    dst = offs[lc] + excl
                    plsc.store_scatter(
                        cidxv, [jnp.full((16,), lc, jnp.int32), dst],
                        idx, mask=m)
                    ptrv[jnp.int32(lc), pl.ds(i16, 16)] = (
                        jnp.where(m, dst, jnp.int32(_CSLOT)))
                    offs[lc] = offs[lc] + plsc.all_reduce_population_count(m)
            return tuple(offs)

        offs = lax.fori_loop(
            jnp.int32(0), jnp.int32(_BLK // 16), cmp_body,
            tuple(z16 for _ in range(5)))
        plsc.subcore_barrier()
        cnts = [jnp.max(offs[k]) for k in range(5)]

        for l in range(2):
            pltpu.make_async_copy(Rs[l].at[idxv.at[jnp.int32(l)]],
                                  rows01.at[jnp.int32(l)], sem).start()
        nchs = []
        for lc in range(5):
            nch = lax.shift_right_logical(cnts[lc] + jnp.int32(127),
                                          jnp.int32(7))
            nchs.append(nch)

            def fire(ck, carry2, lc=lc):
                chunk_copy(lc, ck).start()
                return carry2

            lax.fori_loop(jnp.int32(0), nch, fire, jnp.int32(0))

        for l in range(2):
            pltpu.make_async_copy(Rs[l].at[idxv.at[jnp.int32(l)]],
                                  rows01.at[jnp.int32(l)], sem).wait()
        for lc in range(5):
            def drain(ck, carry2, lc=lc):
                chunk_copy(lc, ck).wait()
                return carry2

            lax.fori_loop(jnp.int32(0), nchs[lc], drain, jnp.int32(0))

        wdescs = []
        for l in range(_NUM_LEVELS):

            def tj_body(tj, carry3, l=l):
                for cb in range(8):
                    p0 = tj * jnp.int32(128) + jnp.int32(cb * 16)
                    if l < 2:
                        pv = lanes + p0
                    elif l < 6:
                        pv = ptrv[jnp.int32(l - 2), pl.ds(p0, 16)]
                    else:
                        pv = ptrv[jnp.int32(4), pl.ds(p0, 16)]
                    for r in range(8):
                        if l < 2:
                            g = plsc.load_gather(
                                rows01,
                                [jnp.full((16,), l, jnp.int32), pv,
                                 jnp.full((16,), r, jnp.int32)])
                        elif l < 6:
                            g = plsc.load_gather(
                                rowsC,
                                [jnp.full((16,), l - 2, jnp.int32), pv,
                                 jnp.full((16,), r, jnp.int32)])
                        else:
                            g = plsc.load_gather(
                                rows67C,
                                [pv, jnp.full((16,), (l - 6) * 8 + r,
                                              jnp.int32)])
                        ostage[jnp.int32(l), tj, jnp.int32(r),
                               pl.ds(cb * 16, 16)] = g
                return carry3

            lax.fori_loop(jnp.int32(0), jnp.int32(_BLK // 128), tj_body,
                          jnp.int32(0))
            tj0 = wid * jnp.int32(_CHUNK // 128) + b * jnp.int32(_BLK // 128)
            d = pltpu.make_async_copy(
                ostage.at[jnp.int32(l)],
                out2.at[jnp.int32(l), pl.ds(tj0, _BLK // 128)], sem2)
            d.start()
            wdescs.append(d)
        for d in wdescs:
            d.wait()
        return carry

    lax.fori_loop(jnp.int32(0), jnp.int32(_NBLK), blk_body, jnp.int32(0))


@jax.jit
def kernel(positions, W0, W1, W2, W3, W4, W5, W6, W7):
    Ws = [W0, W1, W2, W3, W4, W5, W6, W7]
    Wds = [W.reshape(_TAB[i] // 128, 128, 8).transpose(0, 2, 1)
           for i, W in enumerate(Ws)]
    posT = positions.T

    mesh = plsc.VectorSubcoreMesh(**_MESH)

    relayout = pl.kernel(
        _relayout_body,
        out_type=tuple(
            jax.ShapeDtypeStruct((_TAB[i] * _FDIM,), jnp.float32)
            for i in range(6)) + (
            jax.ShapeDtypeStruct((_TAB[6] * 16,), jnp.float32),),
        mesh=mesh,
        scratch_types=[
            pltpu.VMEM((2, 16, 8, 128), jnp.float32),
            pltpu.VMEM((2, 16, 8, 128), jnp.float32),
            pltpu.VMEM((16 * 2048,), jnp.float32),
            pltpu.SemaphoreType.DMA,
        ],
        compiler_params=_CP,
    )
    Rflat = relayout(*Wds)
    Rs = [Rflat[i].reshape(_TAB[i], _FDIM) for i in range(6)]
    R67 = Rflat[6].reshape(_TAB[6], 16)

    lookup = pl.kernel(
        _lookup_body,
        out_type=jax.ShapeDtypeStruct((8, _N // 128, 8, 128), jnp.float32),
        mesh=mesh,
        scratch_types=[
            pltpu.VMEM((3, _BLK), jnp.float32),
            pltpu.VMEM((2, _BLK), jnp.int32),
            pltpu.VMEM((5, _BLK), jnp.int32),
            pltpu.VMEM((5, _BLK), jnp.int32),
            pltpu.VMEM((2, _BLK, _FDIM), jnp.float32),
            pltpu.VMEM((4, _CSLOT + 16, _FDIM), jnp.float32),
            pltpu.VMEM((_CSLOT + 16, 16), jnp.float32),
            pltpu.VMEM((16,), jnp.int32),
            pltpu.VMEM((_NUM_LEVELS, _BLK // 128, 8, 128), jnp.float32),
            pltpu.SemaphoreType.DMA,
            pltpu.SemaphoreType.DMA,
        ],
        compiler_params=_CP,
    )
    out2 = lookup(posT, *Rs, R67)

    return out2.transpose(1, 3, 0, 2).reshape(_N, _NUM_LEVELS * _FDIM)


# speedup vs baseline: 2.8006x; 1.2344x over previous
import jax
import jax.numpy as jnp
from jax import lax
from jax.experimental import pallas as pl
from jax.experimental.pallas import tpu as pltpu
from jax.experimental.pallas import tpu_sc as plsc

_NUM_LEVELS = 8
_BASE_RES = 32
_MAX_RES = 2048
_FDIM = 8
_RES = [min(_BASE_RES * (2 ** l), _MAX_RES) for l in range(_NUM_LEVELS)]
_TAB = [min(r ** 3, 2 ** 19) for r in _RES]
_N = 262144

_NC = 2
_NS = 16
_NW = _NC * _NS
_CHUNK = _N // _NW
_BLK = 512
_NBLK = _CHUNK // _BLK

_CP = pltpu.CompilerParams(needs_layout_passes=False,
                           use_tc_tiling_on_sc=False)
_MESH = dict(core_axis_name="c", subcore_axis_name="s",
             num_cores=_NC, num_subcores=_NS)

_JPW = [t // 128 // _NW for t in _TAB]


def _wid():
    return lax.axis_index("c") * jnp.int32(_NS) + lax.axis_index("s")


def _relayout_body(Wd0, Wd1, Wd2, Wd3, Wd4, Wd5, Wd6, Wd7,
                   R0, R1, R2, R3, R4, R5, R67,
                   buf, buf2, rbuf, sem):
    Wds = [Wd0, Wd1, Wd2, Wd3, Wd4, Wd5, Wd6, Wd7]
    Rs = [R0, R1, R2, R3, R4, R5, R67]
    wid = _wid()
    lanes = lax.iota(jnp.int32, 16)
    rmod = lanes & jnp.int32(7)
    choff = lanes >> jnp.int32(3)
    rl2 = jnp.maximum(lanes - 8, 0)
    mask_lo = lanes < 8
    mask_hi = lanes >= 8

    def do_table(t, pair):
        jpw = _JPW[t]
        cj = min(16, jpw)
        nch = jpw // cj
        Wd = Wds[t]
        R = Rs[t] if not pair else Rs[6]
        fpj = 2048 if pair else 1024

        def read_descs(ck, pp):
            j0 = wid * jnp.int32(jpw) + ck * jnp.int32(cj)
            ds = [pltpu.make_async_copy(
                Wd.at[pl.ds(j0, cj)],
                buf.at[pp, pl.ds(0, cj), slice(None), pl.ds(0, 128)], sem)]
            if pair:
                ds.append(pltpu.make_async_copy(
                    Wds[7].at[pl.ds(j0, cj)],
                    buf2.at[pp, pl.ds(0, cj), slice(None), pl.ds(0, 128)],
                    sem))
            return ds

        for d in read_descs(jnp.int32(0), jnp.int32(0)):
            d.start()

        def chunk_body(ck, carry):
            pp = ck & jnp.int32(1)
            for d in read_descs(ck, pp):
                d.wait()
            nxt = jnp.minimum(ck + jnp.int32(1), jnp.int32(nch - 1))
            for d in read_descs(nxt, pp ^ jnp.int32(1)):
                d.start()
            ppv = jnp.full((16,), 0, jnp.int32) + pp

            def jj_body(jj, carry2):
                jjv = jnp.full((16,), 0, jnp.int32) + jj
                o0 = jj * jnp.int32(fpj)
                if not pair:
                    for k in range(64):
                        cvec = choff + jnp.int32(2 * k)
                        g = plsc.load_gather(buf, [ppv, jjv, rmod, cvec])
                        rbuf[pl.ds(o0 + jnp.int32(16 * k), 16)] = g
                else:
                    for c in range(128):
                        cvec = jnp.full((16,), c, jnp.int32)
                        g = plsc.load_gather(buf, [ppv, jjv, rmod, cvec],
                                             mask=mask_lo)
                        g2 = plsc.load_gather(buf2, [ppv, jjv, rl2, cvec],
                                              mask=mask_hi)
                        g = jnp.where(mask_lo, g, g2)
                        rbuf[pl.ds(o0 + jnp.int32(16 * c), 16)] = g
                return carry2

            lax.fori_loop(jnp.int32(0), jnp.int32(cj), jj_body, jnp.int32(0))
            off = (wid * jnp.int32(jpw) + ck * jnp.int32(cj)) * jnp.int32(fpj)
            pltpu.sync_copy(rbuf.at[pl.ds(0, cj * fpj)],
                            R.at[pl.ds(off, cj * fpj)])
            return carry

        lax.fori_loop(jnp.int32(0), jnp.int32(nch), chunk_body, jnp.int32(0))
        for d in read_descs(jnp.int32(nch - 1), jnp.int32(nch & 1)):
            d.wait()

    for t in range(6):
        do_table(t, pair=False)
    do_table(6, pair=True)


_CSLOT = 512


def _lookup_body(posT, R0, R1, R2, R3, R4, R5, R67, out2,
                 posv, idxv, cidxv, ptrv, rows01, rowsC, rows67C,
                 idxcache, ostage, sem, sem2):
    Rs = [R0, R1, R2, R3, R4, R5]
    wid = _wid()
    base = wid * jnp.int32(_CHUNK)
    lanes = lax.iota(jnp.int32, 16)
    z16 = jnp.zeros((16,), jnp.int32)

    idxcache[pl.ds(0, 16)] = z16 + jnp.int32(_TAB[2] - 1)
    cdescs = []
    for lc in range(4):
        d = pltpu.make_async_copy(
            Rs[lc + 2].at[idxcache.at[pl.ds(0, 8)]],
            rowsC.at[jnp.int32(lc), pl.ds(_CSLOT, 8)], sem)
        d.start()
        cdescs.append(d)
    d = pltpu.make_async_copy(R67.at[idxcache.at[pl.ds(0, 8)]],
                              rows67C.at[pl.ds(_CSLOT, 8)], sem)
    d.start()
    cdescs.append(d)
    for d in cdescs:
        d.wait()

    def chunk_copy(lc, ck):
        cs = ck * jnp.int32(128)
        if lc < 4:
            return pltpu.make_async_copy(
                Rs[lc + 2].at[cidxv.at[jnp.int32(lc), pl.ds(cs, 128)]],
                rowsC.at[jnp.int32(lc), pl.ds(cs, 128)], sem)
        return pltpu.make_async_copy(
            R67.at[cidxv.at[jnp.int32(4), pl.ds(cs, 128)]],
            rows67C.at[pl.ds(cs, 128)], sem)

    def blk_body(b, carry):
        row0 = base + b * jnp.int32(_BLK)
        for f in range(3):
            pltpu.sync_copy(posT.at[jnp.int32(f), pl.ds(row0, _BLK)],
                            posv.at[jnp.int32(f)])

        def zero_body(i, carry2):
            i16 = i * jnp.int32(16)
            for lc in range(5):
                cidxv[jnp.int32(lc), pl.ds(i16, 16)] = z16
            return carry2

        lax.fori_loop(jnp.int32(0), jnp.int32(_BLK // 16), zero_body,
                      jnp.int32(0))

        def cmp_body(i, offs):
            offs = list(offs)
            i16 = i * jnp.int32(16)
            x = posv[jnp.int32(0), pl.ds(i16, 16)]
            y = posv[jnp.int32(1), pl.ds(i16, 16)]
            z = posv[jnp.int32(2), pl.ds(i16, 16)]
            ux = (x + 1.0) * 0.5
            uy = (y + 1.0) * 0.5
            uz = (z + 1.0) * 0.5
            for l in range(7):
                r = float(_RES[l])
                hi = r - 1.0
                px = jnp.minimum(jnp.maximum(ux * r, 0.0), hi)
                py = jnp.minimum(jnp.maximum(uy * r, 0.0), hi)
                pz = jnp.minimum(jnp.maximum(uz * r, 0.0), hi)
                idxf = px * (r * r) + py * r + pz
                idxf = jnp.minimum(idxf, float(_TAB[l] - 1))
                idx = idxf.astype(jnp.int32)
                if l < 2:
                    idxv[jnp.int32(l), pl.ds(i16, 16)] = idx
                else:
                    lc = l - 2
                    m = idx < jnp.int32(_TAB[l] - 1)
                    mi = m.astype(jnp.int32)
                    excl = plsc.cumsum(mi) - mi
                    dst = offs[lc] + excl
                    plsc.store_scatter(
                        cidxv, [jnp.full((16,), lc, jnp.int32), dst],
                        idx, mask=m)
                    ptrv[jnp.int32(lc), pl.ds(i16, 16)] = (
                        jnp.where(m, dst, jnp.int32(_CSLOT)))
                    offs[lc] = offs[lc] + plsc.all_reduce_population_count(m)
            return tuple(offs)

        offs = lax.fori_loop(
            jnp.int32(0), jnp.int32(_BLK // 16), cmp_body,
            tuple(z16 for _ in range(5)))
        plsc.subcore_barrier()
        cnts = [jnp.max(offs[k]) for k in range(5)]

        for l in range(2):
            pltpu.make_async_copy(Rs[l].at[idxv.at[jnp.int32(l)]],
                                  rows01.at[jnp.int32(l)], sem).start()
        nchs = []
        for lc in range(5):
            nch = lax.shift_right_logical(cnts[lc] + jnp.int32(127),
                                          jnp.int32(7))
            nchs.append(nch)

            def fire(ck, carry2, lc=lc):
                chunk_copy(lc, ck).start()
                return carry2

            lax.fori_loop(jnp.int32(0), nch, fire, jnp.int32(0))

        for l in range(2):
            pltpu.make_async_copy(Rs[l].at[idxv.at[jnp.int32(l)]],
                                  rows01.at[jnp.int32(l)], sem).wait()
        for lc in range(5):
            def drain(ck, carry2, lc=lc):
                chunk_copy(lc, ck).wait()
                return carry2

            lax.fori_loop(jnp.int32(0), nchs[lc], drain, jnp.int32(0))

        wdescs = []
        for l in range(_NUM_LEVELS):

            def tj_body(tj, carry3, l=l):
                for cb in range(8):
                    p0 = tj * jnp.int32(128) + jnp.int32(cb * 16)
                    if l < 2:
                        pv = lanes + p0
                    elif l < 6:
                        pv = ptrv[jnp.int32(l - 2), pl.ds(p0, 16)]
                    else:
                        pv = ptrv[jnp.int32(4), pl.ds(p0, 16)]
                    for r in range(8):
                        if l < 2:
                            g = plsc.load_gather(
                                rows01,
                                [jnp.full((16,), l, jnp.int32), pv,
                                 jnp.full((16,), r, jnp.int32)])
                        elif l < 6:
                            g = plsc.load_gather(
                                rowsC,
                                [jnp.full((16,), l - 2, jnp.int32), pv,
                                 jnp.full((16,), r, jnp.int32)])
                        else:
                            g = plsc.load_gather(
                                rows67C,
                                [pv, jnp.full((16,), (l - 6) * 8 + r,
                                              jnp.int32)])
                        ostage[jnp.int32(l), tj, jnp.int32(r),
                               pl.ds(cb * 16, 16)] = g
                return carry3

            lax.fori_loop(jnp.int32(0), jnp.int32(_BLK // 128), tj_body,
                          jnp.int32(0))
            tj0 = wid * jnp.int32(_CHUNK // 128) + b * jnp.int32(_BLK // 128)
            d = pltpu.make_async_copy(
                ostage.at[jnp.int32(l)],
                out2.at[jnp.int32(l), pl.ds(tj0, _BLK // 128)], sem2)
            d.start()
            wdescs.append(d)
        for d in wdescs:
            d.wait()
        return carry

    lax.fori_loop(jnp.int32(0), jnp.int32(_NBLK), blk_body, jnp.int32(0))


@jax.jit
def kernel(positions, W0, W1, W2, W3, W4, W5, W6, W7):
    Ws = [W0, W1, W2, W3, W4, W5, W6, W7]
    Wds = [W.reshape(_TAB[i] // 128, 128, 8).transpose(0, 2, 1)
           for i, W in enumerate(Ws)]
    posT = positions.T

    mesh = plsc.VectorSubcoreMesh(**_MESH)

    relayout = pl.kernel(
        _relayout_body,
        out_type=tuple(
            jax.ShapeDtypeStruct((_TAB[i] * _FDIM,), jnp.float32)
            for i in range(6)) + (
            jax.ShapeDtypeStruct((_TAB[6] * 16,), jnp.float32),),
        mesh=mesh,
        scratch_types=[
            pltpu.VMEM((2, 16, 8, 129), jnp.float32),
            pltpu.VMEM((2, 16, 8, 129), jnp.float32),
            pltpu.VMEM((16 * 2048,), jnp.float32),
            pltpu.SemaphoreType.DMA,
        ],
        compiler_params=_CP,
    )
    Rflat = relayout(*Wds)
    Rs = [Rflat[i].reshape(_TAB[i], _FDIM) for i in range(6)]
    R67 = Rflat[6].reshape(_TAB[6], 16)

    lookup = pl.kernel(
        _lookup_body,
        out_type=jax.ShapeDtypeStruct((8, _N // 128, 8, 128), jnp.float32),
        mesh=mesh,
        scratch_types=[
            pltpu.VMEM((3, _BLK), jnp.float32),
            pltpu.VMEM((2, _BLK), jnp.int32),
            pltpu.VMEM((5, _BLK), jnp.int32),
            pltpu.VMEM((5, _BLK), jnp.int32),
            pltpu.VMEM((2, _BLK, _FDIM), jnp.float32),
            pltpu.VMEM((4, _CSLOT + 16, _FDIM), jnp.float32),
            pltpu.VMEM((_CSLOT + 16, 16), jnp.float32),
            pltpu.VMEM((16,), jnp.int32),
            pltpu.VMEM((_NUM_LEVELS, _BLK // 128, 8, 128), jnp.float32),
            pltpu.SemaphoreType.DMA,
            pltpu.SemaphoreType.DMA,
        ],
        compiler_params=_CP,
    )
    out2 = lookup(posT, *Rs, R67)

    return out2.transpose(1, 3, 0, 2).reshape(_N, _NUM_LEVELS * _FDIM)
